# baseline (speedup 1.0000x reference)
import functools
import numpy as np
import jax
import jax.numpy as jnp
from jax import lax
from jax.experimental import pallas as pl
from jax.experimental.pallas import tpu as pltpu
from jax.experimental.pallas import tpu_sc as plsc

_GROUPED = np.array(
    [0, 8, 16, 24, 32, 40, 48, 56, 1, 9, 17, 25, 33, 41, 49, 57,
     2, 10, 18, 26, 34, 42, 50, 58, 3, 11, 19, 27, 35, 43, 51, 59,
     4, 12, 20, 28, 36, 44, 52, 60, 5, 13, 21, 29, 37, 45, 53, 61,
     6, 14, 22, 30, 38, 46, 54, 62, 7, 15, 23, 31, 39, 47, 55, 63],
    dtype=np.int64)
_RESTORED = _GROUPED.copy()
_PERM = _GROUPED[_RESTORED]


def _contiguous_runs(perm):
    runs = []
    n = len(perm)
    i = 0
    while i < n:
        j = i + 1
        while j < n and perm[j] == perm[j - 1] + 1:
            j += 1
        runs.append((i, int(perm[i]), j - i))
        i = j
    return runs


_RUNS = _contiguous_runs(_PERM)

_NC, _NS = 2, 16
_NW = _NC * _NS

_CB = 2
_NBUF = 2


_LAG = 1


def _sc_body(x_hbm, o_hbm, shared, *scratch):
    sin = scratch[:_NBUF]
    sout = scratch[_NBUF:2 * _NBUF]
    cid = lax.axis_index("c")
    sid = lax.axis_index("s")
    wid = sid * _NC + cid
    nb = x_hbm.shape[0] // _NW
    b0 = wid * nb
    nchunks = nb // _CB
    ngroups = nchunks // _NBUF

    def in_dma(i, s):
        return pltpu.make_async_copy(
            x_hbm.at[pl.ds(b0 + i * _CB, _CB)], shared.at[s, sid], sin[s])

    def out_dmas(i, s):
        return [
            pltpu.make_async_copy(
                shared.at[s, sid, :, pl.ds(src, ln)],
                o_hbm.at[pl.ds(b0 + i * _CB, _CB), pl.ds(dst, ln)],
                sout[s])
            for dst, src, ln in _RUNS
        ]

    def step(i, s, s_lag):
        @pl.when(i >= _NBUF)
        def _():
            for d in out_dmas(i - _NBUF, s):
                d.wait()

        in_dma(i, s).start()
        j = i - _LAG

        @pl.when(j >= 0)
        def _():
            in_dma(j, s_lag).wait()
            for d in out_dmas(j, s_lag):
                d.start()

    def body(g, _):
        for s in range(_NBUF):
            i = g * _NBUF + s
            step(i, s, (s - _LAG) % _NBUF)
        return 0

    lax.fori_loop(0, ngroups, body, 0)
    for j in range(nchunks - _LAG, nchunks):
        s = j % _NBUF
        in_dma(j, s).wait()
        for d in out_dmas(j, s):
            d.start()
    for j in range(nchunks - _NBUF, nchunks):
        s = j % _NBUF
        for d in out_dmas(j, s):
            d.wait()


def kernel(x):
    b, n, c = x.shape
    sc_copy = functools.partial(
        pl.kernel,
        mesh=plsc.VectorSubcoreMesh(core_axis_name="c", subcore_axis_name="s"),
        out_type=jax.ShapeDtypeStruct((b, n, c), x.dtype),
        scratch_types=(
            [pltpu.VMEM_SHARED((_NBUF, _NS, _CB, n, c), jnp.float32)]
            + [pltpu.SemaphoreType.DMA for _ in range(2 * _NBUF)]
        ),
    )(_sc_body)
    return sc_copy(x)


# speedup vs baseline: 1.0051x; 1.0051x over previous
import functools
import numpy as np
import jax
import jax.numpy as jnp
from jax import lax
from jax.experimental import pallas as pl
from jax.experimental.pallas import tpu as pltpu
from jax.experimental.pallas import tpu_sc as plsc

_GROUPED = np.array(
    [0, 8, 16, 24, 32, 40, 48, 56, 1, 9, 17, 25, 33, 41, 49, 57,
     2, 10, 18, 26, 34, 42, 50, 58, 3, 11, 19, 27, 35, 43, 51, 59,
     4, 12, 20, 28, 36, 44, 52, 60, 5, 13, 21, 29, 37, 45, 53, 61,
     6, 14, 22, 30, 38, 46, 54, 62, 7, 15, 23, 31, 39, 47, 55, 63],
    dtype=np.int64)
_RESTORED = _GROUPED.copy()
_PERM = _GROUPED[_RESTORED]


def _contiguous_runs(perm):
    runs = []
    n = len(perm)
    i = 0
    while i < n:
        j = i + 1
        while j < n and perm[j] == perm[j - 1] + 1:
            j += 1
        runs.append((i, int(perm[i]), j - i))
        i = j
    return runs


_RUNS = _contiguous_runs(_PERM)

_NC, _NS = 2, 16
_NW = _NC * _NS

_CB = 2
_NBUF = 2


def _sc_body(x_hbm, o_hbm, shared, *sems):
    sin = sems[:_NBUF]
    sout = sems[_NBUF:2 * _NBUF]
    cid = lax.axis_index("c")
    sid = lax.axis_index("s")
    wid = sid * _NC + cid
    nb = x_hbm.shape[0] // _NW
    b0 = wid * nb
    nchunks = nb // _CB
    ngroups = nchunks // _NBUF

    def in_dma(i, s):
        return pltpu.make_async_copy(
            x_hbm.at[pl.ds(b0 + i * _CB, _CB)], shared.at[s, sid], sin[s])

    def out_dmas(i, s):
        return [
            pltpu.make_async_copy(
                shared.at[s, sid, :, pl.ds(src, ln)],
                o_hbm.at[pl.ds(b0 + i * _CB, _CB), pl.ds(dst, ln)],
                sout[s])
            for dst, src, ln in _RUNS
        ]

    def body(g, _):
        for s in range(_NBUF):
            i = g * _NBUF + s

            @pl.when(g > 0)
            def _():
                for d in out_dmas(i - _NBUF, s):
                    d.wait()

            in_dma(i, s).start()
        for s in range(_NBUF):
            i = g * _NBUF + s
            in_dma(i, s).wait()
            for d in out_dmas(i, s):
                d.start()
        return 0

    lax.fori_loop(0, ngroups, body, 0)
    for s in range(_NBUF):
        i = (ngroups - 1) * _NBUF + s
        for d in out_dmas(i, s):
            d.wait()


def kernel(x):
    b, n, c = x.shape
    sc_copy = functools.partial(
        pl.kernel,
        mesh=plsc.VectorSubcoreMesh(core_axis_name="c", subcore_axis_name="s"),
        out_type=jax.ShapeDtypeStruct((b, n, c), x.dtype),
        scratch_types=(
            [pltpu.VMEM_SHARED((_NBUF, _NS, _CB, n, c), jnp.float32)]
            + [pltpu.SemaphoreType.DMA for _ in range(2 * _NBUF)]
        ),
    )(_sc_body)
    return sc_copy(x)
